# baseline (speedup 1.0000x reference)
import jax
import jax.numpy as jnp
from jax.experimental import pallas as pl

_KEY_HI = 0
_KEY_LO = 42
_INV_KPROB = 2.0

_BLOCK = 196608
_CHUNK = 8192

_ROT_A = (13, 15, 26, 6)
_ROT_B = (17, 29, 16, 24)


def _threefry_keep(idx):
    ks0 = jnp.uint32(_KEY_HI)
    ks1 = jnp.uint32(_KEY_LO)
    ks2 = jnp.uint32(_KEY_HI ^ _KEY_LO ^ 0x1BD11BDA)

    x0 = jnp.zeros(idx.shape, jnp.uint32) + ks0
    x1 = idx + ks1

    def rotl(x, d):
        return (x << jnp.uint32(d)) | (x >> jnp.uint32(32 - d))

    injections = ((ks1, ks2), (ks2, ks0), (ks0, ks1), (ks1, ks2), (ks2, ks0))
    for i, (a, b) in enumerate(injections):
        for r in _ROT_A if i % 2 == 0 else _ROT_B:
            x0 = x0 + x1
            x1 = rotl(x1, r)
            x1 = x1 ^ x0
        x0 = x0 + a
        x1 = x1 + b + jnp.uint32(i + 1)

    bits = x0 ^ x1
    return (bits >> jnp.uint32(31)) == jnp.uint32(1)


def _block_body(values_ref, indices_ref, out_ref, ind_out_ref):
    j = pl.program_id(0)
    rows = _CHUNK // 8
    ind_out_ref[...] = indices_ref[...]
    for c in range(_BLOCK // _CHUNK):
        base = (j * _BLOCK + c * _CHUNK).astype(jnp.uint32)
        idx = (
            base
            + jax.lax.broadcasted_iota(jnp.uint32, (8, rows), 0) * rows
            + jax.lax.broadcasted_iota(jnp.uint32, (8, rows), 1)
        )
        keep = _threefry_keep(idx)
        v = values_ref[c * _CHUNK : (c + 1) * _CHUNK].reshape(8, rows)
        out = jnp.where(keep, v * _INV_KPROB, 0.0)
        out_ref[c * _CHUNK : (c + 1) * _CHUNK] = out.reshape(_CHUNK)


@jax.jit
def _sparse_dropout(indices, values):
    nnz = values.shape[0]
    grid = pl.cdiv(nnz, _BLOCK)
    ind_block = (-(-nnz // grid) + 1023) // 1024 * 1024
    out, ind_out = pl.pallas_call(
        _block_body,
        grid=(grid,),
        in_specs=[
            pl.BlockSpec((_BLOCK,), lambda j: (j,)),
            pl.BlockSpec((2, ind_block), lambda j: (0, j)),
        ],
        out_specs=[
            pl.BlockSpec((_BLOCK,), lambda j: (j,)),
            pl.BlockSpec((2, ind_block), lambda j: (0, j)),
        ],
        out_shape=[
            jax.ShapeDtypeStruct((nnz,), values.dtype),
            jax.ShapeDtypeStruct((2, nnz), indices.dtype),
        ],
    )(values, indices)
    return ind_out, out


def kernel(indices, values):
    ind_out, new_values = _sparse_dropout(indices, values)
    return ind_out, new_values


# speedup vs baseline: 1.0034x; 1.0034x over previous
import jax
import jax.numpy as jnp
from jax.experimental import pallas as pl

_KEY_HI = 0
_KEY_LO = 42
_INV_KPROB = 2.0

_BLOCK = 131072
_CHUNK = 8192

_ROT_A = (13, 15, 26, 6)
_ROT_B = (17, 29, 16, 24)


def _threefry_keep(idx):
    ks0 = jnp.uint32(_KEY_HI)
    ks1 = jnp.uint32(_KEY_LO)
    ks2 = jnp.uint32(_KEY_HI ^ _KEY_LO ^ 0x1BD11BDA)

    x0 = jnp.zeros(idx.shape, jnp.uint32) + ks0
    x1 = idx + ks1

    def rotl(x, d):
        return (x << jnp.uint32(d)) | (x >> jnp.uint32(32 - d))

    injections = ((ks1, ks2), (ks2, ks0), (ks0, ks1), (ks1, ks2), (ks2, ks0))
    for i, (a, b) in enumerate(injections):
        for r in _ROT_A if i % 2 == 0 else _ROT_B:
            x0 = x0 + x1
            x1 = rotl(x1, r)
            x1 = x1 ^ x0
        x0 = x0 + a
        x1 = x1 + b + jnp.uint32(i + 1)

    bits = x0 ^ x1
    return (bits >> jnp.uint32(31)) == jnp.uint32(1)


def _block_body(values_ref, indices_ref, out_ref, ind_out_ref):
    j = pl.program_id(0)
    rows = _CHUNK // 8
    ind_out_ref[...] = indices_ref[...]
    for c in range(_BLOCK // _CHUNK):
        base = (j * _BLOCK + c * _CHUNK).astype(jnp.uint32)
        idx = (
            base
            + jax.lax.broadcasted_iota(jnp.uint32, (8, rows), 0) * rows
            + jax.lax.broadcasted_iota(jnp.uint32, (8, rows), 1)
        )
        keep = _threefry_keep(idx)
        v = values_ref[c * _CHUNK : (c + 1) * _CHUNK].reshape(8, rows)
        out = jnp.where(keep, v * _INV_KPROB, 0.0)
        out_ref[c * _CHUNK : (c + 1) * _CHUNK] = out.reshape(_CHUNK)


@jax.jit
def _sparse_dropout(indices, values):
    nnz = values.shape[0]
    grid = pl.cdiv(nnz, _BLOCK)
    ind_block = (-(-nnz // grid) + 1023) // 1024 * 1024
    out, ind_out = pl.pallas_call(
        _block_body,
        grid=(grid,),
        in_specs=[
            pl.BlockSpec((_BLOCK,), lambda j: (j,)),
            pl.BlockSpec((2, ind_block), lambda j: (0, j)),
        ],
        out_specs=[
            pl.BlockSpec((_BLOCK,), lambda j: (j,)),
            pl.BlockSpec((2, ind_block), lambda j: (0, j)),
        ],
        out_shape=[
            jax.ShapeDtypeStruct((nnz,), values.dtype),
            jax.ShapeDtypeStruct((2, nnz), indices.dtype),
        ],
    )(values, indices)
    return ind_out, out


def kernel(indices, values):
    ind_out, new_values = _sparse_dropout(indices, values)
    return ind_out, new_values
